# baseline (speedup 1.0000x reference)
import functools

import jax
import jax.numpy as jnp
from jax.experimental import pallas as pl
from jax.experimental.pallas import tpu as pltpu

H = 16


def _qkv_body(x_ref, w_ref, b_ref, out_ref):
    x = x_ref[...].astype(jnp.bfloat16)
    acc = jax.lax.dot_general(x, w_ref[...], (((1,), (1,)), ((), ())),
                              preferred_element_type=jnp.float32)
    out_ref[...] = (acc + b_ref[...]).astype(jnp.bfloat16)


def _attn_body(hd, q_ref, k_ref, v_ref, o_ref):
    q = q_ref[0] * jnp.bfloat16(0.125)
    k = k_ref[0]
    v = v_ref[0]
    ones = jnp.ones((v.shape[0], hd), jnp.bfloat16)
    outs = []
    for t in range(q.shape[1] // hd):
        qt = q[:, t * hd:(t + 1) * hd]
        kt = k[:, t * hd:(t + 1) * hd]
        vt = v[:, t * hd:(t + 1) * hd]
        s = jax.lax.dot_general(qt, kt, (((1,), (1,)), ((), ())),
                                preferred_element_type=jnp.float32)
        p = jnp.exp(s.astype(jnp.bfloat16))
        acc = jnp.dot(p, jnp.concatenate([vt, ones], axis=1),
                      preferred_element_type=jnp.float32)
        outs.append((acc[:, :hd] / acc[:, hd:]).astype(jnp.bfloat16))
    o_ref[0] = jnp.concatenate(outs, axis=1)


def _proj_res_body(o_ref, w_ref, b_ref, x_ref, h_ref, s_ref, ss_ref):
    acc = jax.lax.dot_general(o_ref[...], w_ref[...], (((1,), (1,)), ((), ())),
                              preferred_element_type=jnp.float32)
    hval = acc + b_ref[...] + x_ref[...]
    h_ref[...] = hval
    s_ref[...] = jnp.sum(hval, axis=0, keepdims=True)[None]
    ss_ref[...] = jnp.sum(hval * hval, axis=0, keepdims=True)[None]


def _ffn_body(n_rows, h_ref, s1_ref, ss1_ref, g1_ref, bb1_ref,
              w1_ref, b1_ref, w2_ref, b2_ref, z_ref, s2_ref, ss2_ref):
    inv_n = jnp.float32(1.0 / n_rows)
    mean = jnp.sum(s1_ref[...], axis=0) * inv_n
    var = jnp.sum(ss1_ref[...], axis=0) * inv_n - mean * mean
    rstd = jax.lax.rsqrt(var + jnp.float32(1e-5))
    scale = rstd * g1_ref[...]
    y = (h_ref[...] - mean) * scale + bb1_ref[...]
    u = jax.lax.dot_general(y.astype(jnp.bfloat16), w1_ref[...],
                            (((1,), (1,)), ((), ())),
                            preferred_element_type=jnp.float32)
    u = jnp.maximum(u + b1_ref[...], 0.0)
    ff = jax.lax.dot_general(u.astype(jnp.bfloat16), w2_ref[...],
                             (((1,), (1,)), ((), ())),
                             preferred_element_type=jnp.float32)
    z = y + ff + b2_ref[...]
    z_ref[...] = z
    s2_ref[...] = jnp.sum(z, axis=0, keepdims=True)[None]
    ss2_ref[...] = jnp.sum(z * z, axis=0, keepdims=True)[None]


def _bn_body(n_rows, z_ref, s_ref, ss_ref, g_ref, b_ref, out_ref):
    inv_n = jnp.float32(1.0 / n_rows)
    mean = jnp.sum(s_ref[...], axis=0) * inv_n
    var = jnp.sum(ss_ref[...], axis=0) * inv_n - mean * mean
    rstd = jax.lax.rsqrt(var + jnp.float32(1e-5))
    out_ref[...] = (z_ref[...] - mean) * (rstd * g_ref[...]) + b_ref[...]


def _params(nd):
    return pltpu.CompilerParams(dimension_semantics=("parallel",) * nd)


def kernel(x, Wqkv, bqkv, Wo, bo, bn1_g, bn1_b, bn2_g, bn2_b, W1, b1, W2, b2):
    b, s, d = x.shape
    n = b * s
    hd = d // H
    blk = 256
    bq = 512
    n_blk = n // blk

    x_flat = x.reshape(n, d)
    wqkv_bf = Wqkv.astype(jnp.bfloat16)
    wo_bf = Wo.astype(jnp.bfloat16)
    w1_bf = W1.astype(jnp.bfloat16)
    w2_bf = W2.astype(jnp.bfloat16)

    qkv = pl.pallas_call(
        _qkv_body,
        grid=(n_blk,),
        in_specs=[
            pl.BlockSpec((blk, d), lambda i: (i, 0)),
            pl.BlockSpec((3 * d, d), lambda i: (0, 0)),
            pl.BlockSpec((1, 3 * d), lambda i: (0, 0)),
        ],
        out_specs=pl.BlockSpec((blk, 3 * d), lambda i: (i, 0)),
        out_shape=jax.ShapeDtypeStruct((n, 3 * d), jnp.bfloat16),
        compiler_params=_params(1),
    )(x_flat, wqkv_bf, bqkv.reshape(1, 3 * d))
    qkv = qkv.reshape(b, s, 3 * d)

    slab = 128
    n_slab = d // slab
    o = pl.pallas_call(
        functools.partial(_attn_body, hd),
        grid=(b, n_slab, s // bq),
        in_specs=[
            pl.BlockSpec((1, bq, slab), lambda bi, hi, qi: (bi, qi, hi)),
            pl.BlockSpec((1, s, slab), lambda bi, hi, qi: (bi, 0, n_slab + hi)),
            pl.BlockSpec((1, s, slab), lambda bi, hi, qi: (bi, 0, 2 * n_slab + hi)),
        ],
        out_specs=pl.BlockSpec((1, bq, slab), lambda bi, hi, qi: (bi, qi, hi)),
        out_shape=jax.ShapeDtypeStruct((b, s, d), jnp.bfloat16),
        compiler_params=_params(3),
    )(qkv, qkv, qkv)
    o_flat = o.reshape(n, d)

    h1, s1, ss1 = pl.pallas_call(
        _proj_res_body,
        grid=(n_blk,),
        in_specs=[
            pl.BlockSpec((blk, d), lambda i: (i, 0)),
            pl.BlockSpec((d, d), lambda i: (0, 0)),
            pl.BlockSpec((1, d), lambda i: (0, 0)),
            pl.BlockSpec((blk, d), lambda i: (i, 0)),
        ],
        out_specs=[
            pl.BlockSpec((blk, d), lambda i: (i, 0)),
            pl.BlockSpec((1, 1, d), lambda i: (i, 0, 0)),
            pl.BlockSpec((1, 1, d), lambda i: (i, 0, 0)),
        ],
        out_shape=[
            jax.ShapeDtypeStruct((n, d), jnp.float32),
            jax.ShapeDtypeStruct((n_blk, 1, d), jnp.float32),
            jax.ShapeDtypeStruct((n_blk, 1, d), jnp.float32),
        ],
        compiler_params=_params(1),
    )(o_flat, wo_bf, bo.reshape(1, d), x_flat)

    z, s2, ss2 = pl.pallas_call(
        functools.partial(_ffn_body, n),
        grid=(n_blk,),
        in_specs=[
            pl.BlockSpec((blk, d), lambda i: (i, 0)),
            pl.BlockSpec((n_blk, 1, d), lambda i: (0, 0, 0)),
            pl.BlockSpec((n_blk, 1, d), lambda i: (0, 0, 0)),
            pl.BlockSpec((1, d), lambda i: (0, 0)),
            pl.BlockSpec((1, d), lambda i: (0, 0)),
            pl.BlockSpec((2 * d, d), lambda i: (0, 0)),
            pl.BlockSpec((1, 2 * d), lambda i: (0, 0)),
            pl.BlockSpec((d, 2 * d), lambda i: (0, 0)),
            pl.BlockSpec((1, d), lambda i: (0, 0)),
        ],
        out_specs=[
            pl.BlockSpec((blk, d), lambda i: (i, 0)),
            pl.BlockSpec((1, 1, d), lambda i: (i, 0, 0)),
            pl.BlockSpec((1, 1, d), lambda i: (i, 0, 0)),
        ],
        out_shape=[
            jax.ShapeDtypeStruct((n, d), jnp.float32),
            jax.ShapeDtypeStruct((n_blk, 1, d), jnp.float32),
            jax.ShapeDtypeStruct((n_blk, 1, d), jnp.float32),
        ],
        compiler_params=_params(1),
    )(h1, s1, ss1, bn1_g.reshape(1, d), bn1_b.reshape(1, d),
      w1_bf, b1.reshape(1, 2 * d), w2_bf, b2.reshape(1, d))

    out = pl.pallas_call(
        functools.partial(_bn_body, n),
        grid=(n_blk,),
        in_specs=[
            pl.BlockSpec((blk, d), lambda i: (i, 0)),
            pl.BlockSpec((n_blk, 1, d), lambda i: (0, 0, 0)),
            pl.BlockSpec((n_blk, 1, d), lambda i: (0, 0, 0)),
            pl.BlockSpec((1, d), lambda i: (0, 0)),
            pl.BlockSpec((1, d), lambda i: (0, 0)),
        ],
        out_specs=pl.BlockSpec((blk, d), lambda i: (i, 0)),
        out_shape=jax.ShapeDtypeStruct((n, d), jnp.float32),
        compiler_params=_params(1),
    )(z, s2, ss2, bn2_g.reshape(1, d), bn2_b.reshape(1, d))

    return out.reshape(b, s, d)


# speedup vs baseline: 1.0458x; 1.0458x over previous
import functools

import jax
import jax.numpy as jnp
from jax.experimental import pallas as pl
from jax.experimental.pallas import tpu as pltpu

H = 16


def _qkv_body(x_ref, w_ref, b_ref, out_ref):
    x = x_ref[...].astype(jnp.bfloat16)
    acc = jax.lax.dot_general(x, w_ref[...], (((1,), (1,)), ((), ())),
                              preferred_element_type=jnp.float32)
    out_ref[...] = (acc + b_ref[...]).astype(jnp.bfloat16)


def _attn_body(hd, q_ref, k_ref, v_ref, o_ref):
    q = q_ref[0] * jnp.bfloat16(0.125)
    k = k_ref[0]
    v = v_ref[0]
    ones = jnp.ones((v.shape[0], hd), jnp.bfloat16)
    outs = []
    for t in range(q.shape[1] // hd):
        qt = q[:, t * hd:(t + 1) * hd]
        kt = k[:, t * hd:(t + 1) * hd]
        vt = v[:, t * hd:(t + 1) * hd]
        s = jax.lax.dot_general(qt, kt, (((1,), (1,)), ((), ())),
                                preferred_element_type=jnp.float32)
        p = jnp.exp(s.astype(jnp.bfloat16))
        acc = jnp.dot(p, jnp.concatenate([vt, ones], axis=1),
                      preferred_element_type=jnp.float32)
        outs.append((acc[:, :hd] / acc[:, hd:]).astype(jnp.bfloat16))
    o_ref[0] = jnp.concatenate(outs, axis=1)


def _proj_res_body(o_ref, w_ref, b_ref, x_ref, h_ref, s_ref, ss_ref):
    acc = jax.lax.dot_general(o_ref[...], w_ref[...], (((1,), (1,)), ((), ())),
                              preferred_element_type=jnp.float32)
    hval = acc + b_ref[...] + x_ref[...]
    h_ref[...] = hval
    s_ref[...] = jnp.sum(hval, axis=0, keepdims=True)[None]
    ss_ref[...] = jnp.sum(hval * hval, axis=0, keepdims=True)[None]


def _ffn_body(n_rows, h_ref, s1_ref, ss1_ref, g1_ref, bb1_ref,
              w1_ref, b1_ref, w2_ref, b2_ref, z_ref, s2_ref, ss2_ref):
    inv_n = jnp.float32(1.0 / n_rows)
    mean = jnp.sum(s1_ref[...], axis=0) * inv_n
    var = jnp.sum(ss1_ref[...], axis=0) * inv_n - mean * mean
    rstd = jax.lax.rsqrt(var + jnp.float32(1e-5))
    scale = rstd * g1_ref[...]
    y = (h_ref[...] - mean) * scale + bb1_ref[...]
    u = jax.lax.dot_general(y.astype(jnp.bfloat16), w1_ref[...],
                            (((1,), (1,)), ((), ())),
                            preferred_element_type=jnp.float32)
    u = jnp.maximum(u + b1_ref[...], 0.0).astype(jnp.bfloat16)
    ff = jax.lax.dot_general(u, w2_ref[...],
                             (((1,), (1,)), ((), ())),
                             preferred_element_type=jnp.float32)
    z = y + ff + b2_ref[...]
    z_ref[...] = z
    s2_ref[...] = jnp.sum(z, axis=0, keepdims=True)[None]
    ss2_ref[...] = jnp.sum(z * z, axis=0, keepdims=True)[None]


def _bn_body(n_rows, z_ref, s_ref, ss_ref, g_ref, b_ref, out_ref):
    inv_n = jnp.float32(1.0 / n_rows)
    mean = jnp.sum(s_ref[...], axis=0) * inv_n
    var = jnp.sum(ss_ref[...], axis=0) * inv_n - mean * mean
    rstd = jax.lax.rsqrt(var + jnp.float32(1e-5))
    out_ref[...] = (z_ref[...] - mean) * (rstd * g_ref[...]) + b_ref[...]


def _params(nd):
    return pltpu.CompilerParams(dimension_semantics=("parallel",) * nd)


def kernel(x, Wqkv, bqkv, Wo, bo, bn1_g, bn1_b, bn2_g, bn2_b, W1, b1, W2, b2):
    b, s, d = x.shape
    n = b * s
    hd = d // H
    blk = 256
    bq = 1024
    n_blk = n // blk

    x_flat = x.reshape(n, d)
    wqkv_bf = Wqkv.astype(jnp.bfloat16)
    wo_bf = Wo.astype(jnp.bfloat16)
    w1_bf = W1.astype(jnp.bfloat16)
    w2_bf = W2.astype(jnp.bfloat16)

    qkv = pl.pallas_call(
        _qkv_body,
        grid=(n_blk,),
        in_specs=[
            pl.BlockSpec((blk, d), lambda i: (i, 0)),
            pl.BlockSpec((3 * d, d), lambda i: (0, 0)),
            pl.BlockSpec((1, 3 * d), lambda i: (0, 0)),
        ],
        out_specs=pl.BlockSpec((blk, 3 * d), lambda i: (i, 0)),
        out_shape=jax.ShapeDtypeStruct((n, 3 * d), jnp.bfloat16),
        compiler_params=_params(1),
    )(x_flat, wqkv_bf, bqkv.reshape(1, 3 * d))
    qkv = qkv.reshape(b, s, 3 * d)

    slab = 128
    n_slab = d // slab
    o = pl.pallas_call(
        functools.partial(_attn_body, hd),
        grid=(b, n_slab, s // bq),
        in_specs=[
            pl.BlockSpec((1, bq, slab), lambda bi, hi, qi: (bi, qi, hi)),
            pl.BlockSpec((1, s, slab), lambda bi, hi, qi: (bi, 0, n_slab + hi)),
            pl.BlockSpec((1, s, slab), lambda bi, hi, qi: (bi, 0, 2 * n_slab + hi)),
        ],
        out_specs=pl.BlockSpec((1, bq, slab), lambda bi, hi, qi: (bi, qi, hi)),
        out_shape=jax.ShapeDtypeStruct((b, s, d), jnp.bfloat16),
        compiler_params=_params(3),
    )(qkv, qkv, qkv)
    o_flat = o.reshape(n, d)

    h1, s1, ss1 = pl.pallas_call(
        _proj_res_body,
        grid=(n_blk,),
        in_specs=[
            pl.BlockSpec((blk, d), lambda i: (i, 0)),
            pl.BlockSpec((d, d), lambda i: (0, 0)),
            pl.BlockSpec((1, d), lambda i: (0, 0)),
            pl.BlockSpec((blk, d), lambda i: (i, 0)),
        ],
        out_specs=[
            pl.BlockSpec((blk, d), lambda i: (i, 0)),
            pl.BlockSpec((1, 1, d), lambda i: (i, 0, 0)),
            pl.BlockSpec((1, 1, d), lambda i: (i, 0, 0)),
        ],
        out_shape=[
            jax.ShapeDtypeStruct((n, d), jnp.float32),
            jax.ShapeDtypeStruct((n_blk, 1, d), jnp.float32),
            jax.ShapeDtypeStruct((n_blk, 1, d), jnp.float32),
        ],
        compiler_params=_params(1),
    )(o_flat, wo_bf, bo.reshape(1, d), x_flat)

    z, s2, ss2 = pl.pallas_call(
        functools.partial(_ffn_body, n),
        grid=(n_blk,),
        in_specs=[
            pl.BlockSpec((blk, d), lambda i: (i, 0)),
            pl.BlockSpec((n_blk, 1, d), lambda i: (0, 0, 0)),
            pl.BlockSpec((n_blk, 1, d), lambda i: (0, 0, 0)),
            pl.BlockSpec((1, d), lambda i: (0, 0)),
            pl.BlockSpec((1, d), lambda i: (0, 0)),
            pl.BlockSpec((2 * d, d), lambda i: (0, 0)),
            pl.BlockSpec((1, 2 * d), lambda i: (0, 0)),
            pl.BlockSpec((d, 2 * d), lambda i: (0, 0)),
            pl.BlockSpec((1, d), lambda i: (0, 0)),
        ],
        out_specs=[
            pl.BlockSpec((blk, d), lambda i: (i, 0)),
            pl.BlockSpec((1, 1, d), lambda i: (i, 0, 0)),
            pl.BlockSpec((1, 1, d), lambda i: (i, 0, 0)),
        ],
        out_shape=[
            jax.ShapeDtypeStruct((n, d), jnp.float32),
            jax.ShapeDtypeStruct((n_blk, 1, d), jnp.float32),
            jax.ShapeDtypeStruct((n_blk, 1, d), jnp.float32),
        ],
        compiler_params=_params(1),
    )(h1, s1, ss1, bn1_g.reshape(1, d), bn1_b.reshape(1, d),
      w1_bf, b1.reshape(1, 2 * d), w2_bf, b2.reshape(1, d))

    out = pl.pallas_call(
        functools.partial(_bn_body, n),
        grid=(n_blk,),
        in_specs=[
            pl.BlockSpec((blk, d), lambda i: (i, 0)),
            pl.BlockSpec((n_blk, 1, d), lambda i: (0, 0, 0)),
            pl.BlockSpec((n_blk, 1, d), lambda i: (0, 0, 0)),
            pl.BlockSpec((1, d), lambda i: (0, 0)),
            pl.BlockSpec((1, d), lambda i: (0, 0)),
        ],
        out_specs=pl.BlockSpec((blk, d), lambda i: (i, 0)),
        out_shape=jax.ShapeDtypeStruct((n, d), jnp.float32),
        compiler_params=_params(1),
    )(z, s2, ss2, bn2_g.reshape(1, d), bn2_b.reshape(1, d))

    return out.reshape(b, s, d)


# speedup vs baseline: 1.1316x; 1.0820x over previous
import functools

import jax
import jax.numpy as jnp
from jax.experimental import pallas as pl
from jax.experimental.pallas import tpu as pltpu

H = 16


def _qkv_body(x_ref, w_ref, b_ref, out_ref):
    x = x_ref[...].astype(jnp.bfloat16)
    acc = jax.lax.dot_general(x, w_ref[...], (((1,), (1,)), ((), ())),
                              preferred_element_type=jnp.float32)
    out_ref[...] = (acc + b_ref[...]).astype(jnp.bfloat16)


def _attn_body(hd, q_ref, k_ref, v_ref, o_ref):
    q = q_ref[0] * jnp.bfloat16(0.125)
    k = k_ref[0]
    v = v_ref[0]
    ones = jnp.ones((v.shape[0], hd), jnp.bfloat16)
    outs = []
    for t in range(q.shape[1] // hd):
        qt = q[:, t * hd:(t + 1) * hd]
        kt = k[:, t * hd:(t + 1) * hd]
        vt = v[:, t * hd:(t + 1) * hd]
        s = jax.lax.dot_general(qt, kt, (((1,), (1,)), ((), ())),
                                preferred_element_type=jnp.float32)
        p = jnp.exp(s.astype(jnp.bfloat16))
        acc = jnp.dot(p, jnp.concatenate([vt, ones], axis=1),
                      preferred_element_type=jnp.float32)
        outs.append((acc[:, :hd] / acc[:, hd:]).astype(jnp.bfloat16))
    o_ref[0] = jnp.concatenate(outs, axis=1)


def _proj_res_body(o_ref, w_ref, b_ref, x_ref, h_ref, s_ref, ss_ref):
    acc = jax.lax.dot_general(o_ref[...], w_ref[...], (((1,), (1,)), ((), ())),
                              preferred_element_type=jnp.float32)
    hval = acc + b_ref[...] + x_ref[...]
    h_ref[...] = hval
    s_ref[...] = jnp.sum(hval, axis=0, keepdims=True)[None]
    ss_ref[...] = jnp.sum(hval * hval, axis=0, keepdims=True)[None]


def _ffn_body(n_rows, h_ref, s1_ref, ss1_ref, g1_ref, bb1_ref,
              w1_ref, b1_ref, w2_ref, b2_ref, z_ref, s2_ref, ss2_ref):
    inv_n = jnp.float32(1.0 / n_rows)
    mean = jnp.sum(s1_ref[...], axis=0) * inv_n
    var = jnp.sum(ss1_ref[...], axis=0) * inv_n - mean * mean
    rstd = jax.lax.rsqrt(var + jnp.float32(1e-5))
    scale = rstd * g1_ref[...]
    y = (h_ref[...] - mean) * scale + bb1_ref[...]
    u = jax.lax.dot_general(y.astype(jnp.bfloat16), w1_ref[...],
                            (((1,), (1,)), ((), ())),
                            preferred_element_type=jnp.float32)
    u = jnp.maximum(u + b1_ref[...], 0.0).astype(jnp.bfloat16)
    ff = jax.lax.dot_general(u, w2_ref[...],
                             (((1,), (1,)), ((), ())),
                             preferred_element_type=jnp.float32)
    z = y + ff + b2_ref[...]
    z_ref[...] = z
    s2_ref[...] = jnp.sum(z, axis=0, keepdims=True)[None]
    ss2_ref[...] = jnp.sum(z * z, axis=0, keepdims=True)[None]


def _bn_body(n_rows, z_ref, s_ref, ss_ref, g_ref, b_ref, out_ref):
    inv_n = jnp.float32(1.0 / n_rows)
    mean = jnp.sum(s_ref[...], axis=0) * inv_n
    var = jnp.sum(ss_ref[...], axis=0) * inv_n - mean * mean
    rstd = jax.lax.rsqrt(var + jnp.float32(1e-5))
    out_ref[...] = (z_ref[...] - mean) * (rstd * g_ref[...]) + b_ref[...]


def _params(nd):
    return pltpu.CompilerParams(dimension_semantics=("parallel",) * nd)


def kernel(x, Wqkv, bqkv, Wo, bo, bn1_g, bn1_b, bn2_g, bn2_b, W1, b1, W2, b2):
    b, s, d = x.shape
    n = b * s
    hd = d // H
    blk = 512
    bq = 2048
    n_blk = n // blk

    x_flat = x.reshape(n, d)
    wqkv_bf = Wqkv.astype(jnp.bfloat16)
    wo_bf = Wo.astype(jnp.bfloat16)
    w1_bf = W1.astype(jnp.bfloat16)
    w2_bf = W2.astype(jnp.bfloat16)

    qkv = pl.pallas_call(
        _qkv_body,
        grid=(n_blk,),
        in_specs=[
            pl.BlockSpec((blk, d), lambda i: (i, 0)),
            pl.BlockSpec((3 * d, d), lambda i: (0, 0)),
            pl.BlockSpec((1, 3 * d), lambda i: (0, 0)),
        ],
        out_specs=pl.BlockSpec((blk, 3 * d), lambda i: (i, 0)),
        out_shape=jax.ShapeDtypeStruct((n, 3 * d), jnp.bfloat16),
        compiler_params=_params(1),
    )(x_flat, wqkv_bf, bqkv.reshape(1, 3 * d))
    qkv = qkv.reshape(b, s, 3 * d)

    slab = 128
    n_slab = d // slab
    o = pl.pallas_call(
        functools.partial(_attn_body, hd),
        grid=(b, n_slab, s // bq),
        in_specs=[
            pl.BlockSpec((1, bq, slab), lambda bi, hi, qi: (bi, qi, hi)),
            pl.BlockSpec((1, s, slab), lambda bi, hi, qi: (bi, 0, n_slab + hi)),
            pl.BlockSpec((1, s, slab), lambda bi, hi, qi: (bi, 0, 2 * n_slab + hi)),
        ],
        out_specs=pl.BlockSpec((1, bq, slab), lambda bi, hi, qi: (bi, qi, hi)),
        out_shape=jax.ShapeDtypeStruct((b, s, d), jnp.bfloat16),
        compiler_params=_params(3),
    )(qkv, qkv, qkv)
    o_flat = o.reshape(n, d)

    h1, s1, ss1 = pl.pallas_call(
        _proj_res_body,
        grid=(n_blk,),
        in_specs=[
            pl.BlockSpec((blk, d), lambda i: (i, 0)),
            pl.BlockSpec((d, d), lambda i: (0, 0)),
            pl.BlockSpec((1, d), lambda i: (0, 0)),
            pl.BlockSpec((blk, d), lambda i: (i, 0)),
        ],
        out_specs=[
            pl.BlockSpec((blk, d), lambda i: (i, 0)),
            pl.BlockSpec((1, 1, d), lambda i: (i, 0, 0)),
            pl.BlockSpec((1, 1, d), lambda i: (i, 0, 0)),
        ],
        out_shape=[
            jax.ShapeDtypeStruct((n, d), jnp.float32),
            jax.ShapeDtypeStruct((n_blk, 1, d), jnp.float32),
            jax.ShapeDtypeStruct((n_blk, 1, d), jnp.float32),
        ],
        compiler_params=_params(1),
    )(o_flat, wo_bf, bo.reshape(1, d), x_flat)

    z, s2, ss2 = pl.pallas_call(
        functools.partial(_ffn_body, n),
        grid=(n_blk,),
        in_specs=[
            pl.BlockSpec((blk, d), lambda i: (i, 0)),
            pl.BlockSpec((n_blk, 1, d), lambda i: (0, 0, 0)),
            pl.BlockSpec((n_blk, 1, d), lambda i: (0, 0, 0)),
            pl.BlockSpec((1, d), lambda i: (0, 0)),
            pl.BlockSpec((1, d), lambda i: (0, 0)),
            pl.BlockSpec((2 * d, d), lambda i: (0, 0)),
            pl.BlockSpec((1, 2 * d), lambda i: (0, 0)),
            pl.BlockSpec((d, 2 * d), lambda i: (0, 0)),
            pl.BlockSpec((1, d), lambda i: (0, 0)),
        ],
        out_specs=[
            pl.BlockSpec((blk, d), lambda i: (i, 0)),
            pl.BlockSpec((1, 1, d), lambda i: (i, 0, 0)),
            pl.BlockSpec((1, 1, d), lambda i: (i, 0, 0)),
        ],
        out_shape=[
            jax.ShapeDtypeStruct((n, d), jnp.float32),
            jax.ShapeDtypeStruct((n_blk, 1, d), jnp.float32),
            jax.ShapeDtypeStruct((n_blk, 1, d), jnp.float32),
        ],
        compiler_params=_params(1),
    )(h1, s1, ss1, bn1_g.reshape(1, d), bn1_b.reshape(1, d),
      w1_bf, b1.reshape(1, 2 * d), w2_bf, b2.reshape(1, d))

    out = pl.pallas_call(
        functools.partial(_bn_body, n),
        grid=(n_blk,),
        in_specs=[
            pl.BlockSpec((blk, d), lambda i: (i, 0)),
            pl.BlockSpec((n_blk, 1, d), lambda i: (0, 0, 0)),
            pl.BlockSpec((n_blk, 1, d), lambda i: (0, 0, 0)),
            pl.BlockSpec((1, d), lambda i: (0, 0)),
            pl.BlockSpec((1, d), lambda i: (0, 0)),
        ],
        out_specs=pl.BlockSpec((blk, d), lambda i: (i, 0)),
        out_shape=jax.ShapeDtypeStruct((n, d), jnp.float32),
        compiler_params=_params(1),
    )(z, s2, ss2, bn2_g.reshape(1, d), bn2_b.reshape(1, d))

    return out.reshape(b, s, d)


# speedup vs baseline: 1.3943x; 1.2322x over previous
import functools

import jax
import jax.numpy as jnp
from jax.experimental import pallas as pl
from jax.experimental.pallas import tpu as pltpu

H = 16


def _qkv_body(x_ref, w_ref, b_ref, out_ref):
    x = x_ref[...].astype(jnp.bfloat16)
    acc = jax.lax.dot_general(x, w_ref[...], (((1,), (1,)), ((), ())),
                              preferred_element_type=jnp.float32)
    out_ref[...] = (acc + b_ref[...]).astype(jnp.bfloat16)


def _attn_body(hd, q_ref, k_ref, v_ref, o_ref):
    f8 = jnp.float8_e4m3fn
    q = (q_ref[0] * jnp.bfloat16(0.125)).astype(f8)
    k = k_ref[0].astype(f8)
    v = v_ref[0].astype(f8)
    ones = jnp.ones((v.shape[0], hd), f8)
    outs = []
    for t in range(q.shape[1] // hd):
        qt = q[:, t * hd:(t + 1) * hd]
        kt = k[:, t * hd:(t + 1) * hd]
        vt = v[:, t * hd:(t + 1) * hd]
        s = jax.lax.dot_general(qt, kt, (((1,), (1,)), ((), ())),
                                preferred_element_type=jnp.float32)
        p = jnp.exp(s.astype(jnp.bfloat16)).astype(f8)
        acc = jnp.dot(p, jnp.concatenate([vt, ones], axis=1),
                      preferred_element_type=jnp.float32)
        outs.append((acc[:, :hd] / acc[:, hd:]).astype(jnp.bfloat16))
    o_ref[0] = jnp.concatenate(outs, axis=1)


def _proj_res_body(o_ref, w_ref, b_ref, x_ref, h_ref, s_ref, ss_ref):
    acc = jax.lax.dot_general(o_ref[...], w_ref[...], (((1,), (1,)), ((), ())),
                              preferred_element_type=jnp.float32)
    hval = acc + b_ref[...] + x_ref[...]
    h_ref[...] = hval
    s_ref[...] = jnp.sum(hval, axis=0, keepdims=True)[None]
    ss_ref[...] = jnp.sum(hval * hval, axis=0, keepdims=True)[None]


def _ffn_body(n_rows, h_ref, s1_ref, ss1_ref, g1_ref, bb1_ref,
              w1_ref, b1_ref, w2_ref, b2_ref, z_ref, s2_ref, ss2_ref):
    inv_n = jnp.float32(1.0 / n_rows)
    mean = jnp.sum(s1_ref[...], axis=0) * inv_n
    var = jnp.sum(ss1_ref[...], axis=0) * inv_n - mean * mean
    rstd = jax.lax.rsqrt(var + jnp.float32(1e-5))
    scale = rstd * g1_ref[...]
    y = (h_ref[...] - mean) * scale + bb1_ref[...]
    u = jax.lax.dot_general(y.astype(jnp.bfloat16), w1_ref[...],
                            (((1,), (1,)), ((), ())),
                            preferred_element_type=jnp.float32)
    u = jnp.maximum(u + b1_ref[...], 0.0).astype(jnp.bfloat16)
    ff = jax.lax.dot_general(u, w2_ref[...],
                             (((1,), (1,)), ((), ())),
                             preferred_element_type=jnp.float32)
    z = y + ff + b2_ref[...]
    z_ref[...] = z
    s2_ref[...] = jnp.sum(z, axis=0, keepdims=True)[None]
    ss2_ref[...] = jnp.sum(z * z, axis=0, keepdims=True)[None]


def _bn_body(n_rows, z_ref, s_ref, ss_ref, g_ref, b_ref, out_ref):
    inv_n = jnp.float32(1.0 / n_rows)
    mean = jnp.sum(s_ref[...], axis=0) * inv_n
    var = jnp.sum(ss_ref[...], axis=0) * inv_n - mean * mean
    rstd = jax.lax.rsqrt(var + jnp.float32(1e-5))
    out_ref[...] = (z_ref[...] - mean) * (rstd * g_ref[...]) + b_ref[...]


def _params(nd):
    return pltpu.CompilerParams(dimension_semantics=("parallel",) * nd)


def kernel(x, Wqkv, bqkv, Wo, bo, bn1_g, bn1_b, bn2_g, bn2_b, W1, b1, W2, b2):
    b, s, d = x.shape
    n = b * s
    hd = d // H
    blk = 512
    bq = 2048
    n_blk = n // blk

    x_flat = x.reshape(n, d)
    wqkv_bf = Wqkv.astype(jnp.bfloat16)
    wo_bf = Wo.astype(jnp.bfloat16)
    w1_bf = W1.astype(jnp.bfloat16)
    w2_bf = W2.astype(jnp.bfloat16)

    qkv = pl.pallas_call(
        _qkv_body,
        grid=(n_blk,),
        in_specs=[
            pl.BlockSpec((blk, d), lambda i: (i, 0)),
            pl.BlockSpec((3 * d, d), lambda i: (0, 0)),
            pl.BlockSpec((1, 3 * d), lambda i: (0, 0)),
        ],
        out_specs=pl.BlockSpec((blk, 3 * d), lambda i: (i, 0)),
        out_shape=jax.ShapeDtypeStruct((n, 3 * d), jnp.bfloat16),
        compiler_params=_params(1),
    )(x_flat, wqkv_bf, bqkv.reshape(1, 3 * d))
    qkv = qkv.reshape(b, s, 3 * d)

    slab = 128
    n_slab = d // slab
    o = pl.pallas_call(
        functools.partial(_attn_body, hd),
        grid=(b, n_slab, s // bq),
        in_specs=[
            pl.BlockSpec((1, bq, slab), lambda bi, hi, qi: (bi, qi, hi)),
            pl.BlockSpec((1, s, slab), lambda bi, hi, qi: (bi, 0, n_slab + hi)),
            pl.BlockSpec((1, s, slab), lambda bi, hi, qi: (bi, 0, 2 * n_slab + hi)),
        ],
        out_specs=pl.BlockSpec((1, bq, slab), lambda bi, hi, qi: (bi, qi, hi)),
        out_shape=jax.ShapeDtypeStruct((b, s, d), jnp.bfloat16),
        compiler_params=_params(3),
    )(qkv, qkv, qkv)
    o_flat = o.reshape(n, d)

    h1, s1, ss1 = pl.pallas_call(
        _proj_res_body,
        grid=(n_blk,),
        in_specs=[
            pl.BlockSpec((blk, d), lambda i: (i, 0)),
            pl.BlockSpec((d, d), lambda i: (0, 0)),
            pl.BlockSpec((1, d), lambda i: (0, 0)),
            pl.BlockSpec((blk, d), lambda i: (i, 0)),
        ],
        out_specs=[
            pl.BlockSpec((blk, d), lambda i: (i, 0)),
            pl.BlockSpec((1, 1, d), lambda i: (i, 0, 0)),
            pl.BlockSpec((1, 1, d), lambda i: (i, 0, 0)),
        ],
        out_shape=[
            jax.ShapeDtypeStruct((n, d), jnp.float32),
            jax.ShapeDtypeStruct((n_blk, 1, d), jnp.float32),
            jax.ShapeDtypeStruct((n_blk, 1, d), jnp.float32),
        ],
        compiler_params=_params(1),
    )(o_flat, wo_bf, bo.reshape(1, d), x_flat)

    z, s2, ss2 = pl.pallas_call(
        functools.partial(_ffn_body, n),
        grid=(n_blk,),
        in_specs=[
            pl.BlockSpec((blk, d), lambda i: (i, 0)),
            pl.BlockSpec((n_blk, 1, d), lambda i: (0, 0, 0)),
            pl.BlockSpec((n_blk, 1, d), lambda i: (0, 0, 0)),
            pl.BlockSpec((1, d), lambda i: (0, 0)),
            pl.BlockSpec((1, d), lambda i: (0, 0)),
            pl.BlockSpec((2 * d, d), lambda i: (0, 0)),
            pl.BlockSpec((1, 2 * d), lambda i: (0, 0)),
            pl.BlockSpec((d, 2 * d), lambda i: (0, 0)),
            pl.BlockSpec((1, d), lambda i: (0, 0)),
        ],
        out_specs=[
            pl.BlockSpec((blk, d), lambda i: (i, 0)),
            pl.BlockSpec((1, 1, d), lambda i: (i, 0, 0)),
            pl.BlockSpec((1, 1, d), lambda i: (i, 0, 0)),
        ],
        out_shape=[
            jax.ShapeDtypeStruct((n, d), jnp.float32),
            jax.ShapeDtypeStruct((n_blk, 1, d), jnp.float32),
            jax.ShapeDtypeStruct((n_blk, 1, d), jnp.float32),
        ],
        compiler_params=_params(1),
    )(h1, s1, ss1, bn1_g.reshape(1, d), bn1_b.reshape(1, d),
      w1_bf, b1.reshape(1, 2 * d), w2_bf, b2.reshape(1, d))

    out = pl.pallas_call(
        functools.partial(_bn_body, n),
        grid=(n_blk,),
        in_specs=[
            pl.BlockSpec((blk, d), lambda i: (i, 0)),
            pl.BlockSpec((n_blk, 1, d), lambda i: (0, 0, 0)),
            pl.BlockSpec((n_blk, 1, d), lambda i: (0, 0, 0)),
            pl.BlockSpec((1, d), lambda i: (0, 0)),
            pl.BlockSpec((1, d), lambda i: (0, 0)),
        ],
        out_specs=pl.BlockSpec((blk, d), lambda i: (i, 0)),
        out_shape=jax.ShapeDtypeStruct((n, d), jnp.float32),
        compiler_params=_params(1),
    )(z, s2, ss2, bn2_g.reshape(1, d), bn2_b.reshape(1, d))

    return out.reshape(b, s, d)


# speedup vs baseline: 1.4499x; 1.0399x over previous
import functools

import jax
import jax.numpy as jnp
from jax.experimental import pallas as pl
from jax.experimental.pallas import tpu as pltpu

H = 16


def _qkv_body(x_ref, w_ref, b_ref, out_ref):
    x = x_ref[...].astype(jnp.bfloat16)
    acc = jax.lax.dot_general(x, w_ref[...], (((1,), (1,)), ((), ())),
                              preferred_element_type=jnp.float32)
    out_ref[...] = (acc + b_ref[...]).astype(jnp.bfloat16)


def _attn_body(hd, q_ref, k_ref, v_ref, o_ref):
    f8 = jnp.float8_e4m3fn
    q = (q_ref[0] * jnp.bfloat16(0.125)).astype(f8)
    k = k_ref[0].astype(f8)
    v = v_ref[0].astype(f8)
    ones = jnp.ones((v.shape[0], hd), f8)
    outs = []
    for t in range(q.shape[1] // hd):
        qt = q[:, t * hd:(t + 1) * hd]
        kt = k[:, t * hd:(t + 1) * hd]
        vt = v[:, t * hd:(t + 1) * hd]
        s = jax.lax.dot_general(qt, kt, (((1,), (1,)), ((), ())),
                                preferred_element_type=jnp.float32)
        p = jnp.exp(s.astype(jnp.bfloat16)).astype(f8)
        acc = jnp.dot(p, jnp.concatenate([vt, ones], axis=1),
                      preferred_element_type=jnp.float32)
        outs.append((acc[:, :hd] / acc[:, hd:]).astype(jnp.bfloat16))
    o_ref[0] = jnp.concatenate(outs, axis=1)


def _proj_res_body(o_ref, w_ref, b_ref, x_ref, h_ref, s_ref, ss_ref):
    acc = jax.lax.dot_general(o_ref[...], w_ref[...], (((1,), (1,)), ((), ())),
                              preferred_element_type=jnp.float32)
    hval = acc + b_ref[...] + x_ref[...]
    h_ref[...] = hval.astype(jnp.bfloat16)
    s_ref[...] = jnp.sum(hval, axis=0, keepdims=True)[None]
    ss_ref[...] = jnp.sum(hval * hval, axis=0, keepdims=True)[None]


def _ffn_body(n_rows, h_ref, s1_ref, ss1_ref, g1_ref, bb1_ref,
              w1_ref, b1_ref, w2_ref, b2_ref, z_ref, s2_ref, ss2_ref):
    inv_n = jnp.float32(1.0 / n_rows)
    mean = jnp.sum(s1_ref[...], axis=0) * inv_n
    var = jnp.sum(ss1_ref[...], axis=0) * inv_n - mean * mean
    rstd = jax.lax.rsqrt(var + jnp.float32(1e-5))
    scale = rstd * g1_ref[...]
    y = (h_ref[...].astype(jnp.float32) - mean) * scale + bb1_ref[...]
    u = jax.lax.dot_general(y.astype(jnp.bfloat16), w1_ref[...],
                            (((1,), (1,)), ((), ())),
                            preferred_element_type=jnp.float32)
    u = jnp.maximum(u + b1_ref[...], 0.0).astype(jnp.bfloat16)
    ff = jax.lax.dot_general(u, w2_ref[...],
                             (((1,), (1,)), ((), ())),
                             preferred_element_type=jnp.float32)
    z = y + ff + b2_ref[...]
    z_ref[...] = z.astype(jnp.bfloat16)
    s2_ref[...] = jnp.sum(z, axis=0, keepdims=True)[None]
    ss2_ref[...] = jnp.sum(z * z, axis=0, keepdims=True)[None]


def _bn_body(n_rows, z_ref, s_ref, ss_ref, g_ref, b_ref, out_ref):
    inv_n = jnp.float32(1.0 / n_rows)
    mean = jnp.sum(s_ref[...], axis=0) * inv_n
    var = jnp.sum(ss_ref[...], axis=0) * inv_n - mean * mean
    rstd = jax.lax.rsqrt(var + jnp.float32(1e-5))
    out_ref[...] = (z_ref[...].astype(jnp.float32) - mean) * (rstd * g_ref[...]) + b_ref[...]


def _params(nd):
    return pltpu.CompilerParams(dimension_semantics=("parallel",) * nd)


def kernel(x, Wqkv, bqkv, Wo, bo, bn1_g, bn1_b, bn2_g, bn2_b, W1, b1, W2, b2):
    b, s, d = x.shape
    n = b * s
    hd = d // H
    blk = 1024
    bq = 2048
    n_blk = n // blk

    x_flat = x.reshape(n, d)
    wqkv_bf = Wqkv.astype(jnp.bfloat16)
    wo_bf = Wo.astype(jnp.bfloat16)
    w1_bf = W1.astype(jnp.bfloat16)
    w2_bf = W2.astype(jnp.bfloat16)

    qkv = pl.pallas_call(
        _qkv_body,
        grid=(n_blk,),
        in_specs=[
            pl.BlockSpec((blk, d), lambda i: (i, 0)),
            pl.BlockSpec((3 * d, d), lambda i: (0, 0)),
            pl.BlockSpec((1, 3 * d), lambda i: (0, 0)),
        ],
        out_specs=pl.BlockSpec((blk, 3 * d), lambda i: (i, 0)),
        out_shape=jax.ShapeDtypeStruct((n, 3 * d), jnp.bfloat16),
        compiler_params=_params(1),
    )(x_flat, wqkv_bf, bqkv.reshape(1, 3 * d))
    qkv = qkv.reshape(b, s, 3 * d)

    slab = 128
    n_slab = d // slab
    o = pl.pallas_call(
        functools.partial(_attn_body, hd),
        grid=(b, n_slab, s // bq),
        in_specs=[
            pl.BlockSpec((1, bq, slab), lambda bi, hi, qi: (bi, qi, hi)),
            pl.BlockSpec((1, s, slab), lambda bi, hi, qi: (bi, 0, n_slab + hi)),
            pl.BlockSpec((1, s, slab), lambda bi, hi, qi: (bi, 0, 2 * n_slab + hi)),
        ],
        out_specs=pl.BlockSpec((1, bq, slab), lambda bi, hi, qi: (bi, qi, hi)),
        out_shape=jax.ShapeDtypeStruct((b, s, d), jnp.bfloat16),
        compiler_params=_params(3),
    )(qkv, qkv, qkv)
    o_flat = o.reshape(n, d)

    h1, s1, ss1 = pl.pallas_call(
        _proj_res_body,
        grid=(n_blk,),
        in_specs=[
            pl.BlockSpec((blk, d), lambda i: (i, 0)),
            pl.BlockSpec((d, d), lambda i: (0, 0)),
            pl.BlockSpec((1, d), lambda i: (0, 0)),
            pl.BlockSpec((blk, d), lambda i: (i, 0)),
        ],
        out_specs=[
            pl.BlockSpec((blk, d), lambda i: (i, 0)),
            pl.BlockSpec((1, 1, d), lambda i: (i, 0, 0)),
            pl.BlockSpec((1, 1, d), lambda i: (i, 0, 0)),
        ],
        out_shape=[
            jax.ShapeDtypeStruct((n, d), jnp.bfloat16),
            jax.ShapeDtypeStruct((n_blk, 1, d), jnp.float32),
            jax.ShapeDtypeStruct((n_blk, 1, d), jnp.float32),
        ],
        compiler_params=_params(1),
    )(o_flat, wo_bf, bo.reshape(1, d), x_flat)

    z, s2, ss2 = pl.pallas_call(
        functools.partial(_ffn_body, n),
        grid=(n_blk,),
        in_specs=[
            pl.BlockSpec((blk, d), lambda i: (i, 0)),
            pl.BlockSpec((n_blk, 1, d), lambda i: (0, 0, 0)),
            pl.BlockSpec((n_blk, 1, d), lambda i: (0, 0, 0)),
            pl.BlockSpec((1, d), lambda i: (0, 0)),
            pl.BlockSpec((1, d), lambda i: (0, 0)),
            pl.BlockSpec((2 * d, d), lambda i: (0, 0)),
            pl.BlockSpec((1, 2 * d), lambda i: (0, 0)),
            pl.BlockSpec((d, 2 * d), lambda i: (0, 0)),
            pl.BlockSpec((1, d), lambda i: (0, 0)),
        ],
        out_specs=[
            pl.BlockSpec((blk, d), lambda i: (i, 0)),
            pl.BlockSpec((1, 1, d), lambda i: (i, 0, 0)),
            pl.BlockSpec((1, 1, d), lambda i: (i, 0, 0)),
        ],
        out_shape=[
            jax.ShapeDtypeStruct((n, d), jnp.bfloat16),
            jax.ShapeDtypeStruct((n_blk, 1, d), jnp.float32),
            jax.ShapeDtypeStruct((n_blk, 1, d), jnp.float32),
        ],
        compiler_params=_params(1),
    )(h1, s1, ss1, bn1_g.reshape(1, d), bn1_b.reshape(1, d),
      w1_bf, b1.reshape(1, 2 * d), w2_bf, b2.reshape(1, d))

    out = pl.pallas_call(
        functools.partial(_bn_body, n),
        grid=(n_blk,),
        in_specs=[
            pl.BlockSpec((blk, d), lambda i: (i, 0)),
            pl.BlockSpec((n_blk, 1, d), lambda i: (0, 0, 0)),
            pl.BlockSpec((n_blk, 1, d), lambda i: (0, 0, 0)),
            pl.BlockSpec((1, d), lambda i: (0, 0)),
            pl.BlockSpec((1, d), lambda i: (0, 0)),
        ],
        out_specs=pl.BlockSpec((blk, d), lambda i: (i, 0)),
        out_shape=jax.ShapeDtypeStruct((n, d), jnp.float32),
        compiler_params=_params(1),
    )(z, s2, ss2, bn2_g.reshape(1, d), bn2_b.reshape(1, d))

    return out.reshape(b, s, d)
